# baseline (speedup 1.0000x reference)
import functools

import jax
import jax.numpy as jnp
from jax import lax
from jax.experimental import pallas as pl
from jax.experimental.pallas import tpu as pltpu
from jax.experimental.pallas import tpu_sc as plsc

NUM_CODES = 512
LATENT_DIM = 384
BETA = 0.25
N_TOKENS = 32768
BN = 1024
NSTRIPE = 4
ST = N_TOKENS // NSTRIPE
G_S = ST // BN

_SC_INFO = plsc.get_sparse_core_info()
NC = _SC_INFO.num_cores
NS = _SC_INFO.num_subcores
NW = NC * NS
RPW = ST // NW
CH = 128
NCHUNK = RPW // CH


def _tc_body(z_ref, wn_ref, s_ref, idx_ref, loss_ref):
    i = pl.program_id(0)

    @pl.when(i == 0)
    def _init():
        loss_ref[...] = jnp.zeros((1, 1), jnp.float32)

    wn = wn_ref[...]
    z = z_ref[...]
    zn = z / jnp.maximum(jnp.sqrt(jnp.sum(z * z, axis=1, keepdims=True)), 1e-12)
    t = jnp.sum(zn * zn, axis=1, keepdims=True)
    m = jax.lax.dot_general(zn, wn, (((1,), (1,)), ((), ())),
                            preferred_element_type=jnp.float32)
    d = (t + s_ref[...]) - 2.0 * m
    mind = jnp.min(d, axis=1)
    iota = jax.lax.broadcasted_iota(jnp.int32, d.shape, 1)
    idx = jnp.min(jnp.where(d == mind[:, None], iota, NUM_CODES), axis=1)
    idx_ref[0, 0, :] = idx
    loss_ref[...] += jnp.sum(mind)[None, None]


def _tc_stripe(k):
    return pl.pallas_call(
        _tc_body,
        grid=(G_S,),
        in_specs=[
            pl.BlockSpec((BN, LATENT_DIM), lambda i, k=k: (k * G_S + i, 0)),
            pl.BlockSpec((NUM_CODES, LATENT_DIM), lambda i: (0, 0)),
            pl.BlockSpec((1, NUM_CODES), lambda i: (0, 0)),
        ],
        out_specs=[
            pl.BlockSpec((1, 1, BN), lambda i: (i, 0, 0)),
            pl.BlockSpec((1, 1), lambda i: (0, 0)),
        ],
        out_shape=[
            jax.ShapeDtypeStruct((G_S, 1, BN), jnp.int32),
            jax.ShapeDtypeStruct((1, 1), jnp.float32),
        ],
        compiler_params=pltpu.CompilerParams(
            dimension_semantics=("arbitrary",),
        ),
    )


_sc_mesh = plsc.VectorSubcoreMesh(core_axis_name="c", subcore_axis_name="s")


@functools.partial(
    pl.kernel,
    mesh=_sc_mesh,
    out_type=jax.ShapeDtypeStruct((ST, LATENT_DIM), jnp.float32),
    scratch_types=[
        pltpu.VMEM((CH,), jnp.int32),
        pltpu.VMEM((CH,), jnp.int32),
        pltpu.VMEM((CH, LATENT_DIM), jnp.float32),
        pltpu.VMEM((CH, LATENT_DIM), jnp.float32),
        pltpu.SemaphoreType.DMA,
        pltpu.SemaphoreType.DMA,
    ],
)
def _sc_gather(wn_hbm, idx_hbm, out_hbm, idx_v0, idx_v1, rows_v0, rows_v1,
               sem0, sem1):
    wid = lax.axis_index("s") * NC + lax.axis_index("c")
    base = wid * RPW
    idx_bufs = (idx_v0, idx_v1)
    row_bufs = (rows_v0, rows_v1)
    sems = (sem0, sem1)
    handles = [None, None]
    pltpu.sync_copy(idx_hbm.at[pl.ds(base, CH)], idx_bufs[0])
    handles[0] = pltpu.async_copy(wn_hbm.at[idx_bufs[0]], row_bufs[0], sems[0])
    for j in range(NCHUNK):
        b, nb = j % 2, (j + 1) % 2
        if j + 1 < NCHUNK:
            off = base + (j + 1) * CH
            pltpu.sync_copy(idx_hbm.at[pl.ds(off, CH)], idx_bufs[nb])
            handles[nb] = pltpu.async_copy(
                wn_hbm.at[idx_bufs[nb]], row_bufs[nb], sems[nb])
        handles[b].wait()
        pltpu.sync_copy(row_bufs[b], out_hbm.at[pl.ds(base + j * CH, CH)])


@jax.jit
def _run(z, weight):
    wn = weight / jnp.maximum(jnp.linalg.norm(weight, axis=1, keepdims=True), 1e-12)
    s = jnp.sum(wn ** 2, axis=1)
    idx_parts, zq_parts, loss_parts = [], [], []
    for k in range(NSTRIPE):
        idx3, losssum = _tc_stripe(k)(z, wn, s[None, :])
        idx_k = idx3.reshape(ST)
        zq_parts.append(_sc_gather(wn, idx_k))
        idx_parts.append(idx_k)
        loss_parts.append(losssum)
    zq = jnp.concatenate(zq_parts, axis=0)
    idx = jnp.concatenate(idx_parts, axis=0)
    losssum = sum(p[0, 0] for p in loss_parts)
    loss = losssum * ((1.0 + BETA) / (N_TOKENS * LATENT_DIM))
    return zq, idx, loss


def kernel(z, weight):
    return _run(z, weight)


# speedup vs baseline: 1.1676x; 1.1676x over previous
import functools

import jax
import jax.numpy as jnp
from jax import lax
from jax.experimental import pallas as pl
from jax.experimental.pallas import tpu as pltpu
from jax.experimental.pallas import tpu_sc as plsc

NUM_CODES = 512
LATENT_DIM = 384
BETA = 0.25
N_TOKENS = 32768
BN = 1024
NSTRIPE = 1
ST = N_TOKENS // NSTRIPE
G_S = ST // BN

_SC_INFO = plsc.get_sparse_core_info()
NC = _SC_INFO.num_cores
NS = _SC_INFO.num_subcores
NW = NC * NS
RPW = ST // NW
CH = 128
NCHUNK = RPW // CH


def _tc_body(z_ref, wn_ref, s_ref, idx_ref, loss_ref):
    i = pl.program_id(0)

    @pl.when(i == 0)
    def _init():
        loss_ref[...] = jnp.zeros((1, 1), jnp.float32)

    wn = wn_ref[...]
    z = z_ref[...]
    zn = z / jnp.maximum(jnp.sqrt(jnp.sum(z * z, axis=1, keepdims=True)), 1e-12)
    t = jnp.sum(zn * zn, axis=1, keepdims=True)
    m = jax.lax.dot_general(zn, wn, (((1,), (1,)), ((), ())),
                            preferred_element_type=jnp.float32)
    d = (t + s_ref[...]) - 2.0 * m
    mind = jnp.min(d, axis=1)
    iota = jax.lax.broadcasted_iota(jnp.int32, d.shape, 1)
    idx = jnp.min(jnp.where(d == mind[:, None], iota, NUM_CODES), axis=1)
    idx_ref[0, 0, :] = idx
    loss_ref[...] += jnp.sum(mind)[None, None]


def _tc_stripe(k):
    return pl.pallas_call(
        _tc_body,
        grid=(G_S,),
        in_specs=[
            pl.BlockSpec((BN, LATENT_DIM), lambda i, k=k: (k * G_S + i, 0)),
            pl.BlockSpec((NUM_CODES, LATENT_DIM), lambda i: (0, 0)),
            pl.BlockSpec((1, NUM_CODES), lambda i: (0, 0)),
        ],
        out_specs=[
            pl.BlockSpec((1, 1, BN), lambda i: (i, 0, 0)),
            pl.BlockSpec((1, 1), lambda i: (0, 0)),
        ],
        out_shape=[
            jax.ShapeDtypeStruct((G_S, 1, BN), jnp.int32),
            jax.ShapeDtypeStruct((1, 1), jnp.float32),
        ],
        compiler_params=pltpu.CompilerParams(
            dimension_semantics=("arbitrary",),
        ),
    )


_sc_mesh = plsc.VectorSubcoreMesh(core_axis_name="c", subcore_axis_name="s")


@functools.partial(
    pl.kernel,
    mesh=_sc_mesh,
    out_type=jax.ShapeDtypeStruct((ST, LATENT_DIM), jnp.float32),
    scratch_types=[
        pltpu.VMEM((CH,), jnp.int32),
        pltpu.VMEM((CH,), jnp.int32),
        pltpu.VMEM((CH, LATENT_DIM), jnp.float32),
        pltpu.VMEM((CH, LATENT_DIM), jnp.float32),
        pltpu.SemaphoreType.DMA,
        pltpu.SemaphoreType.DMA,
    ],
)
def _sc_gather(wn_hbm, idx_hbm, out_hbm, idx_v0, idx_v1, rows_v0, rows_v1,
               sem0, sem1):
    wid = lax.axis_index("s") * NC + lax.axis_index("c")
    base = wid * RPW
    idx_bufs = (idx_v0, idx_v1)
    row_bufs = (rows_v0, rows_v1)
    sems = (sem0, sem1)
    handles = [None, None]
    pltpu.sync_copy(idx_hbm.at[pl.ds(base, CH)], idx_bufs[0])
    handles[0] = pltpu.async_copy(wn_hbm.at[idx_bufs[0]], row_bufs[0], sems[0])
    for j in range(NCHUNK):
        b, nb = j % 2, (j + 1) % 2
        if j + 1 < NCHUNK:
            off = base + (j + 1) * CH
            pltpu.sync_copy(idx_hbm.at[pl.ds(off, CH)], idx_bufs[nb])
            handles[nb] = pltpu.async_copy(
                wn_hbm.at[idx_bufs[nb]], row_bufs[nb], sems[nb])
        handles[b].wait()
        pltpu.sync_copy(row_bufs[b], out_hbm.at[pl.ds(base + j * CH, CH)])


@jax.jit
def _run(z, weight):
    wn = weight / jnp.maximum(jnp.linalg.norm(weight, axis=1, keepdims=True), 1e-12)
    s = jnp.sum(wn ** 2, axis=1)
    idx_parts, zq_parts, loss_parts = [], [], []
    for k in range(NSTRIPE):
        idx3, losssum = _tc_stripe(k)(z, wn, s[None, :])
        idx_k = idx3.reshape(ST)
        zq_parts.append(_sc_gather(wn, idx_k))
        idx_parts.append(idx_k)
        loss_parts.append(losssum)
    zq = zq_parts[0] if NSTRIPE == 1 else jnp.concatenate(zq_parts, axis=0)
    idx = idx_parts[0] if NSTRIPE == 1 else jnp.concatenate(idx_parts, axis=0)
    losssum = sum(p[0, 0] for p in loss_parts)
    loss = losssum * ((1.0 + BETA) / (N_TOKENS * LATENT_DIM))
    return zq, idx, loss


def kernel(z, weight):
    return _run(z, weight)


# speedup vs baseline: 1.3283x; 1.1376x over previous
import functools

import jax
import jax.numpy as jnp
from jax import lax
from jax.experimental import pallas as pl
from jax.experimental.pallas import tpu as pltpu
from jax.experimental.pallas import tpu_sc as plsc

NUM_CODES = 512
LATENT_DIM = 384
BETA = 0.25
N_TOKENS = 32768
BN = 2048
NSTRIPE = 1
ST = N_TOKENS // NSTRIPE
G_S = ST // BN

_SC_INFO = plsc.get_sparse_core_info()
NC = _SC_INFO.num_cores
NS = _SC_INFO.num_subcores
NW = NC * NS
RPW = ST // NW
CH = 128
NCHUNK = RPW // CH


def _tc_body(z_ref, wn_ref, s_ref, idx_ref, loss_ref):
    i = pl.program_id(0)

    @pl.when(i == 0)
    def _init():
        loss_ref[...] = jnp.zeros((1, 1), jnp.float32)

    wn = wn_ref[...]
    z = z_ref[...]
    zn = z / jnp.maximum(jnp.sqrt(jnp.sum(z * z, axis=1, keepdims=True)), 1e-12)
    t = jnp.sum(zn * zn, axis=1, keepdims=True)
    m = jax.lax.dot_general(zn, wn, (((1,), (1,)), ((), ())),
                            preferred_element_type=jnp.float32)
    d = (t + s_ref[...]) - 2.0 * m
    mind = jnp.min(d, axis=1)
    iota = jax.lax.broadcasted_iota(jnp.int32, d.shape, 1)
    idx = jnp.min(jnp.where(d == mind[:, None], iota, NUM_CODES), axis=1)
    idx_ref[0, 0, :] = idx
    loss_ref[...] += jnp.sum(mind)[None, None]


def _tc_stripe(k):
    return pl.pallas_call(
        _tc_body,
        grid=(G_S,),
        in_specs=[
            pl.BlockSpec((BN, LATENT_DIM), lambda i, k=k: (k * G_S + i, 0)),
            pl.BlockSpec((NUM_CODES, LATENT_DIM), lambda i: (0, 0)),
            pl.BlockSpec((1, NUM_CODES), lambda i: (0, 0)),
        ],
        out_specs=[
            pl.BlockSpec((1, 1, BN), lambda i: (i, 0, 0)),
            pl.BlockSpec((1, 1), lambda i: (0, 0)),
        ],
        out_shape=[
            jax.ShapeDtypeStruct((G_S, 1, BN), jnp.int32),
            jax.ShapeDtypeStruct((1, 1), jnp.float32),
        ],
        compiler_params=pltpu.CompilerParams(
            dimension_semantics=("arbitrary",),
        ),
    )


_sc_mesh = plsc.VectorSubcoreMesh(core_axis_name="c", subcore_axis_name="s")


@functools.partial(
    pl.kernel,
    mesh=_sc_mesh,
    out_type=jax.ShapeDtypeStruct((ST, LATENT_DIM), jnp.float32),
    scratch_types=[
        pltpu.VMEM((CH,), jnp.int32),
        pltpu.VMEM((CH,), jnp.int32),
        pltpu.VMEM((CH, LATENT_DIM), jnp.float32),
        pltpu.VMEM((CH, LATENT_DIM), jnp.float32),
        pltpu.SemaphoreType.DMA,
        pltpu.SemaphoreType.DMA,
    ],
)
def _sc_gather(wn_hbm, idx_hbm, out_hbm, idx_v0, idx_v1, rows_v0, rows_v1,
               sem0, sem1):
    wid = lax.axis_index("s") * NC + lax.axis_index("c")
    base = wid * RPW
    idx_bufs = (idx_v0, idx_v1)
    row_bufs = (rows_v0, rows_v1)
    sems = (sem0, sem1)
    handles = [None, None]
    pltpu.sync_copy(idx_hbm.at[pl.ds(base, CH)], idx_bufs[0])
    handles[0] = pltpu.async_copy(wn_hbm.at[idx_bufs[0]], row_bufs[0], sems[0])
    for j in range(NCHUNK):
        b, nb = j % 2, (j + 1) % 2
        if j + 1 < NCHUNK:
            off = base + (j + 1) * CH
            pltpu.sync_copy(idx_hbm.at[pl.ds(off, CH)], idx_bufs[nb])
            handles[nb] = pltpu.async_copy(
                wn_hbm.at[idx_bufs[nb]], row_bufs[nb], sems[nb])
        handles[b].wait()
        pltpu.sync_copy(row_bufs[b], out_hbm.at[pl.ds(base + j * CH, CH)])


@jax.jit
def _run(z, weight):
    wn = weight / jnp.maximum(jnp.linalg.norm(weight, axis=1, keepdims=True), 1e-12)
    s = jnp.sum(wn ** 2, axis=1)
    idx_parts, zq_parts, loss_parts = [], [], []
    for k in range(NSTRIPE):
        idx3, losssum = _tc_stripe(k)(z, wn, s[None, :])
        idx_k = idx3.reshape(ST)
        zq_parts.append(_sc_gather(wn, idx_k))
        idx_parts.append(idx_k)
        loss_parts.append(losssum)
    zq = zq_parts[0] if NSTRIPE == 1 else jnp.concatenate(zq_parts, axis=0)
    idx = idx_parts[0] if NSTRIPE == 1 else jnp.concatenate(idx_parts, axis=0)
    losssum = sum(p[0, 0] for p in loss_parts)
    loss = losssum * ((1.0 + BETA) / (N_TOKENS * LATENT_DIM))
    return zq, idx, loss


def kernel(z, weight):
    return _run(z, weight)


# speedup vs baseline: 1.4001x; 1.0541x over previous
import functools

import jax
import jax.numpy as jnp
from jax import lax
from jax.experimental import pallas as pl
from jax.experimental.pallas import tpu as pltpu
from jax.experimental.pallas import tpu_sc as plsc

NUM_CODES = 512
LATENT_DIM = 384
BETA = 0.25
N_TOKENS = 32768
BN = 4096
NSTRIPE = 1
ST = N_TOKENS // NSTRIPE
G_S = ST // BN

_SC_INFO = plsc.get_sparse_core_info()
NC = _SC_INFO.num_cores
NS = _SC_INFO.num_subcores
NW = NC * NS
RPW = ST // NW
CH = 128
NCHUNK = RPW // CH


def _tc_body(z_ref, wn_ref, s_ref, idx_ref, loss_ref):
    i = pl.program_id(0)

    @pl.when(i == 0)
    def _init():
        loss_ref[...] = jnp.zeros((1, 1), jnp.float32)

    wn = wn_ref[...]
    z = z_ref[...]
    zn = z / jnp.maximum(jnp.sqrt(jnp.sum(z * z, axis=1, keepdims=True)), 1e-12)
    t = jnp.sum(zn * zn, axis=1, keepdims=True)
    m = jax.lax.dot_general(zn, wn, (((1,), (1,)), ((), ())),
                            preferred_element_type=jnp.float32)
    d = (t + s_ref[...]) - 2.0 * m
    mind = jnp.min(d, axis=1)
    iota = jax.lax.broadcasted_iota(jnp.int32, d.shape, 1)
    idx = jnp.min(jnp.where(d == mind[:, None], iota, NUM_CODES), axis=1)
    idx_ref[0, 0, :] = idx
    loss_ref[...] += jnp.sum(mind)[None, None]


def _tc_stripe(k):
    return pl.pallas_call(
        _tc_body,
        grid=(G_S,),
        in_specs=[
            pl.BlockSpec((BN, LATENT_DIM), lambda i, k=k: (k * G_S + i, 0)),
            pl.BlockSpec((NUM_CODES, LATENT_DIM), lambda i: (0, 0)),
            pl.BlockSpec((1, NUM_CODES), lambda i: (0, 0)),
        ],
        out_specs=[
            pl.BlockSpec((1, 1, BN), lambda i: (i, 0, 0)),
            pl.BlockSpec((1, 1), lambda i: (0, 0)),
        ],
        out_shape=[
            jax.ShapeDtypeStruct((G_S, 1, BN), jnp.int32),
            jax.ShapeDtypeStruct((1, 1), jnp.float32),
        ],
        compiler_params=pltpu.CompilerParams(
            dimension_semantics=("arbitrary",),
        ),
    )


_sc_mesh = plsc.VectorSubcoreMesh(core_axis_name="c", subcore_axis_name="s")


@functools.partial(
    pl.kernel,
    mesh=_sc_mesh,
    out_type=jax.ShapeDtypeStruct((ST, LATENT_DIM), jnp.float32),
    scratch_types=[
        pltpu.VMEM((CH,), jnp.int32),
        pltpu.VMEM((CH,), jnp.int32),
        pltpu.VMEM((CH, LATENT_DIM), jnp.float32),
        pltpu.VMEM((CH, LATENT_DIM), jnp.float32),
        pltpu.SemaphoreType.DMA,
        pltpu.SemaphoreType.DMA,
    ],
)
def _sc_gather(wn_hbm, idx_hbm, out_hbm, idx_v0, idx_v1, rows_v0, rows_v1,
               sem0, sem1):
    wid = lax.axis_index("s") * NC + lax.axis_index("c")
    base = wid * RPW
    idx_bufs = (idx_v0, idx_v1)
    row_bufs = (rows_v0, rows_v1)
    sems = (sem0, sem1)
    handles = [None, None]
    pltpu.sync_copy(idx_hbm.at[pl.ds(base, CH)], idx_bufs[0])
    handles[0] = pltpu.async_copy(wn_hbm.at[idx_bufs[0]], row_bufs[0], sems[0])
    for j in range(NCHUNK):
        b, nb = j % 2, (j + 1) % 2
        if j + 1 < NCHUNK:
            off = base + (j + 1) * CH
            pltpu.sync_copy(idx_hbm.at[pl.ds(off, CH)], idx_bufs[nb])
            handles[nb] = pltpu.async_copy(
                wn_hbm.at[idx_bufs[nb]], row_bufs[nb], sems[nb])
        handles[b].wait()
        pltpu.sync_copy(row_bufs[b], out_hbm.at[pl.ds(base + j * CH, CH)])


@jax.jit
def _run(z, weight):
    wn = weight / jnp.maximum(jnp.linalg.norm(weight, axis=1, keepdims=True), 1e-12)
    s = jnp.sum(wn ** 2, axis=1)
    idx_parts, zq_parts, loss_parts = [], [], []
    for k in range(NSTRIPE):
        idx3, losssum = _tc_stripe(k)(z, wn, s[None, :])
        idx_k = idx3.reshape(ST)
        zq_parts.append(_sc_gather(wn, idx_k))
        idx_parts.append(idx_k)
        loss_parts.append(losssum)
    zq = zq_parts[0] if NSTRIPE == 1 else jnp.concatenate(zq_parts, axis=0)
    idx = idx_parts[0] if NSTRIPE == 1 else jnp.concatenate(idx_parts, axis=0)
    losssum = sum(p[0, 0] for p in loss_parts)
    loss = losssum * ((1.0 + BETA) / (N_TOKENS * LATENT_DIM))
    return zq, idx, loss


def kernel(z, weight):
    return _run(z, weight)
